# baseline (speedup 1.0000x reference)
import functools

import jax
import jax.numpy as jnp
from jax.experimental import pallas as pl
from jax.experimental.pallas import tpu as pltpu

_K = 8


def _se_step(x_ref, mseg_ref, p_ref, g_ref, w1_ref, b1_ref, w2_ref, b2_ref,
             o_ref, *, inv_hw):
    x2 = x_ref[0]
    mseg = mseg_ref[...]
    p = p_ref[...]
    g = g_ref[...]
    r, _ = x2.shape
    c = g.shape[1]

    sums = jax.lax.dot_general(x2, mseg, (((1,), (1,)), ((), ())),
                               preferred_element_type=jnp.float32)
    t = jax.lax.dot_general(sums, p, (((1,), (0,)), ((), ())),
                            preferred_element_type=jnp.float32)
    m = jnp.sum(t * g, axis=0, keepdims=True) * inv_hw

    h = jax.lax.dot_general(m, w1_ref[...], (((1,), (1,)), ((), ())),
                            preferred_element_type=jnp.float32)
    h = jnp.maximum(h + b1_ref[...], 0.0)
    z = jax.lax.dot_general(h, w2_ref[...], (((1,), (1,)), ((), ())),
                            preferred_element_type=jnp.float32)
    s = jax.nn.sigmoid(z + b2_ref[...])

    ss = jnp.broadcast_to(s, (r, c)) * g
    s8 = jax.lax.dot_general(ss, p, (((1,), (1,)), ((), ())),
                             preferred_element_type=jnp.float32)
    scale = jax.lax.dot_general(s8, mseg, (((1,), (0,)), ((), ())),
                                preferred_element_type=jnp.float32)
    o_ref[0] = (x2 * scale.astype(x2.dtype)).astype(o_ref.dtype)


def kernel(x, w1, b1, w2, b2):
    B, C, H, W = x.shape
    Cr = w1.shape[0]
    HW = H * W
    R = C // _K
    L = _K * HW

    x_flat = x.reshape(B, R, L)
    b1r = b1.astype(jnp.float32).reshape(1, Cr)
    b2r = b2.astype(jnp.float32).reshape(1, C)
    w1f = w1.astype(jnp.float32)
    w2f = w2.astype(jnp.float32)

    iota = jax.lax.broadcasted_iota
    mseg = (iota(jnp.int32, (_K, L), 1) // HW
            == iota(jnp.int32, (_K, L), 0)).astype(jnp.float32)
    pmat = (jnp.remainder(iota(jnp.int32, (_K, C), 1), _K)
            == iota(jnp.int32, (_K, C), 0)).astype(jnp.float32)
    gmat = (iota(jnp.int32, (R, C), 1) // _K
            == iota(jnp.int32, (R, C), 0)).astype(jnp.float32)

    out_flat = pl.pallas_call(
        functools.partial(_se_step, inv_hw=1.0 / HW),
        out_shape=jax.ShapeDtypeStruct((B, R, L), x.dtype),
        grid=(B,),
        in_specs=[
            pl.BlockSpec((1, R, L), lambda b: (b, 0, 0)),
            pl.BlockSpec((_K, L), lambda b: (0, 0)),
            pl.BlockSpec((_K, C), lambda b: (0, 0)),
            pl.BlockSpec((R, C), lambda b: (0, 0)),
            pl.BlockSpec((Cr, C), lambda b: (0, 0)),
            pl.BlockSpec((1, Cr), lambda b: (0, 0)),
            pl.BlockSpec((C, Cr), lambda b: (0, 0)),
            pl.BlockSpec((1, C), lambda b: (0, 0)),
        ],
        out_specs=pl.BlockSpec((1, R, L), lambda b: (b, 0, 0)),
        compiler_params=pltpu.CompilerParams(
            dimension_semantics=("parallel",),
            vmem_limit_bytes=56 << 20,
        ),
        cost_estimate=pl.CostEstimate(
            flops=int(2 * B * C * HW * (_K + 1) + 4 * B * C * Cr),
            transcendentals=int(B * C),
            bytes_accessed=int(2 * B * C * HW * 4),
        ),
    )(x_flat, mseg, pmat, gmat, w1f, b1r, w2f, b2r)

    return out_flat.reshape(B, C, H, W)


# speedup vs baseline: 2.6741x; 2.6741x over previous
import functools

import jax
import jax.numpy as jnp
from jax.experimental import pallas as pl
from jax.experimental.pallas import tpu as pltpu

_NBUF = 3


def _se_compute(x, w1_ref, b1_ref, w2_ref, b2_ref, inv_hw):
    m = jnp.sum(x, axis=-1) * inv_hw
    h = jax.lax.dot_general(m, w1_ref[...], (((1,), (1,)), ((), ())),
                            preferred_element_type=jnp.float32)
    h = jnp.maximum(h + b1_ref[...], 0.0)
    z = jax.lax.dot_general(h, w2_ref[...], (((1,), (1,)), ((), ())),
                            preferred_element_type=jnp.float32)
    s = jax.nn.sigmoid(z + b2_ref[...])
    return (x * s[:, :, None].astype(x.dtype)).astype(x.dtype)


def _se_kernel(x_hbm, w1_ref, b1_ref, w2_ref, b2_ref, o_hbm,
               ibufs, obufs, isems, osems, *, nbatch, inv_hw):

    def start_in(b):
        slot = jax.lax.rem(b, _NBUF)
        pltpu.make_async_copy(x_hbm.at[b], ibufs.at[slot], isems.at[slot]
                              ).start()

    for b in range(min(_NBUF, nbatch)):
        start_in(b)

    def body(b, carry):
        slot = jax.lax.rem(b, _NBUF)
        pltpu.make_async_copy(ibufs.at[slot], ibufs.at[slot], isems.at[slot]
                              ).wait()
        res = _se_compute(ibufs[slot], w1_ref, b1_ref, w2_ref, b2_ref, inv_hw)

        @pl.when(b >= _NBUF)
        def _():
            pltpu.make_async_copy(obufs.at[slot], obufs.at[slot],
                                  osems.at[slot]).wait()

        obufs[slot] = res
        pltpu.make_async_copy(obufs.at[slot], o_hbm.at[b], osems.at[slot]
                              ).start()

        @pl.when(b + _NBUF < nbatch)
        def _():
            start_in(b + _NBUF)

        return carry

    jax.lax.fori_loop(0, nbatch, body, 0)

    for t in range(min(_NBUF, nbatch)):
        b = nbatch - 1 - t
        slot = jax.lax.rem(b, _NBUF)
        pltpu.make_async_copy(obufs.at[slot], obufs.at[slot], osems.at[slot]
                              ).wait()


def kernel(x, w1, b1, w2, b2):
    B, C, H, W = x.shape
    Cr = w1.shape[0]
    HW = H * W

    x_flat = x.reshape(B, 1, C, HW)
    b1r = b1.astype(jnp.float32).reshape(1, Cr)
    b2r = b2.astype(jnp.float32).reshape(1, C)
    w1f = w1.astype(jnp.float32)
    w2f = w2.astype(jnp.float32)

    out_flat = pl.pallas_call(
        functools.partial(_se_kernel, nbatch=B, inv_hw=1.0 / HW),
        out_shape=jax.ShapeDtypeStruct((B, 1, C, HW), x.dtype),
        in_specs=[
            pl.BlockSpec(memory_space=pl.ANY),
            pl.BlockSpec((Cr, C), lambda: (0, 0)),
            pl.BlockSpec((1, Cr), lambda: (0, 0)),
            pl.BlockSpec((C, Cr), lambda: (0, 0)),
            pl.BlockSpec((1, C), lambda: (0, 0)),
        ],
        out_specs=pl.BlockSpec(memory_space=pl.ANY),
        scratch_shapes=[
            pltpu.VMEM((_NBUF, 1, C, HW), x.dtype),
            pltpu.VMEM((_NBUF, 1, C, HW), x.dtype),
            pltpu.SemaphoreType.DMA((_NBUF,)),
            pltpu.SemaphoreType.DMA((_NBUF,)),
        ],
        compiler_params=pltpu.CompilerParams(
            vmem_limit_bytes=56 << 20,
        ),
        cost_estimate=pl.CostEstimate(
            flops=int(2 * B * C * HW + 4 * B * C * Cr),
            transcendentals=int(B * C),
            bytes_accessed=int(2 * B * C * HW * 4),
        ),
    )(x_flat, w1f, b1r, w2f, b2r)

    return out_flat.reshape(B, C, H, W)
